# Initial kernel scaffold; baseline (speedup 1.0000x reference)
import jax
import jax.numpy as jnp
from jax.experimental import pallas as pl


def kernel(x_user, x_item, edge_index_follows, edge_index_clicks, W_follows, b_follows, W_clicks, b_clicks):
    raise NotImplementedError("write your pallas kernel here")


# speedup vs baseline: 9.6827x; 9.6827x over previous
import jax
import jax.numpy as jnp
from jax import lax
from jax.experimental import pallas as pl
from jax.experimental.pallas import tpu as pltpu
from jax.experimental.pallas import tpu_sc as plsc

N_NODES = 10000
E_EDGES = 320000
D = 128

NUM_CORES = 2
NUM_SUBCORES = 16
CHUNK = 125
EDGES_PER_TILE = E_EDGES // NUM_SUBCORES
NCHUNK = EDGES_PER_TILE // CHUNK
NPAIR = NCHUNK // 2
ROWS_PER_TILE = N_NODES // NUM_SUBCORES
CNT_W = 8


def _sc_aggregate_body(x_hbm, idx_hbm, zrow_hbm, zcnt_hbm, ones_hbm,
                       acc_hbm, cnt_hbm,
                       idxa, idxb, msga, msgb, ones_v,
                       sem_a, sem_b, sem_ia, sem_ib,
                       acc_sh, cnt_sh):
    c = lax.axis_index("c")
    s = lax.axis_index("s")
    rows = pl.ds(s * ROWS_PER_TILE, ROWS_PER_TILE)

    pltpu.sync_copy(zrow_hbm, acc_sh.at[rows, :])
    pltpu.sync_copy(zcnt_hbm, cnt_sh.at[rows, :])
    pltpu.sync_copy(ones_hbm, ones_v)
    plsc.subcore_barrier()

    pltpu.sync_copy(idx_hbm.at[c, s, 0], idxa)
    pltpu.async_copy(idx_hbm.at[c, s, 1], idxb, sem_ib)
    pltpu.async_copy(x_hbm.at[idxa.at[0]], msga, sem_a)

    def step(i, carry):
        j0 = 2 * i
        pltpu.make_async_copy(idx_hbm.at[c, s, j0 + 1], idxb, sem_ib).wait()
        pltpu.make_async_copy(x_hbm.at[idxa.at[0]], msga, sem_a).wait()
        pltpu.async_copy(x_hbm.at[idxb.at[0]], msgb, sem_b)
        pltpu.sync_copy(msga, acc_sh.at[idxa.at[1]], add=True)
        pltpu.sync_copy(ones_v, cnt_sh.at[idxa.at[1]], add=True)

        @pl.when(i + 1 < NPAIR)
        def _():
            pltpu.async_copy(idx_hbm.at[c, s, j0 + 2], idxa, sem_ia)

        pltpu.make_async_copy(x_hbm.at[idxb.at[0]], msgb, sem_b).wait()
        pltpu.sync_copy(msgb, acc_sh.at[idxb.at[1]], add=True)
        pltpu.sync_copy(ones_v, cnt_sh.at[idxb.at[1]], add=True)

        @pl.when(i + 1 < NPAIR)
        def _():
            pltpu.make_async_copy(idx_hbm.at[c, s, j0 + 2], idxa, sem_ia).wait()
            pltpu.async_copy(x_hbm.at[idxa.at[0]], msga, sem_a)
            pltpu.async_copy(idx_hbm.at[c, s, j0 + 3], idxb, sem_ib)

        return carry

    lax.fori_loop(0, NPAIR, step, 0)
    plsc.subcore_barrier()

    pltpu.sync_copy(acc_sh.at[rows, :], acc_hbm.at[c, rows, :])
    pltpu.sync_copy(cnt_sh.at[rows, :], cnt_hbm.at[c, rows, :])


def _tc_linear_body(acc_ref, cnt_ref, wt_ref, b_ref, out_ref):
    cnt = cnt_ref[0, :, 0:1]
    recip = 1.0 / jnp.maximum(cnt, 1.0)
    gate = jnp.minimum(cnt, 1.0)
    x = acc_ref[0] * recip
    out_ref[0] = (jnp.dot(x, wt_ref[0], preferred_element_type=jnp.float32)
                  + gate * b_ref[0])


def kernel(x_user, x_item, edge_index_follows, edge_index_clicks,
           W_follows, b_follows, W_clicks, b_clicks):
    del x_item

    def _chunked(ei):
        return ei.reshape(2, NUM_SUBCORES, NCHUNK, CHUNK).transpose(1, 2, 0, 3)

    idx = jnp.stack([_chunked(edge_index_follows), _chunked(edge_index_clicks)])
    zrow = jnp.zeros((ROWS_PER_TILE, D), jnp.float32)
    zcnt = jnp.zeros((ROWS_PER_TILE, CNT_W), jnp.float32)
    ones = jnp.ones((CHUNK, CNT_W), jnp.float32)

    mesh = plsc.VectorSubcoreMesh(core_axis_name="c", subcore_axis_name="s",
                                  num_cores=NUM_CORES,
                                  num_subcores=NUM_SUBCORES)
    acc, cnt = pl.kernel(
        _sc_aggregate_body,
        out_type=[
            jax.ShapeDtypeStruct((NUM_CORES, N_NODES, D), jnp.float32),
            jax.ShapeDtypeStruct((NUM_CORES, N_NODES, CNT_W), jnp.float32),
        ],
        mesh=mesh,
        compiler_params=pltpu.CompilerParams(use_tc_tiling_on_sc=False),
        scratch_types=[
            pltpu.VMEM((2, CHUNK), jnp.int32),
            pltpu.VMEM((2, CHUNK), jnp.int32),
            pltpu.VMEM((CHUNK, D), jnp.float32),
            pltpu.VMEM((CHUNK, D), jnp.float32),
            pltpu.VMEM((CHUNK, CNT_W), jnp.float32),
            pltpu.SemaphoreType.DMA,
            pltpu.SemaphoreType.DMA,
            pltpu.SemaphoreType.DMA,
            pltpu.SemaphoreType.DMA,
            pltpu.VMEM_SHARED((N_NODES, D), jnp.float32),
            pltpu.VMEM_SHARED((N_NODES, CNT_W), jnp.float32),
        ],
    )(x_user, idx, zrow, zcnt, ones)

    wt = jnp.stack([W_follows.T, W_clicks.T])
    b = jnp.stack([b_follows, b_clicks]).reshape(NUM_CORES, 1, D)

    ROW_BLK = 1000
    out = pl.pallas_call(
        _tc_linear_body,
        grid=(NUM_CORES, N_NODES // ROW_BLK),
        in_specs=[
            pl.BlockSpec((1, ROW_BLK, D), lambda t, m: (t, m, 0)),
            pl.BlockSpec((1, ROW_BLK, CNT_W), lambda t, m: (t, m, 0)),
            pl.BlockSpec((1, D, D), lambda t, m: (t, 0, 0)),
            pl.BlockSpec((1, 1, D), lambda t, m: (t, 0, 0)),
        ],
        out_specs=pl.BlockSpec((1, ROW_BLK, D), lambda t, m: (t, m, 0)),
        out_shape=jax.ShapeDtypeStruct((NUM_CORES, N_NODES, D), jnp.float32),
    )(acc, cnt, wt, b)

    return (out[0], out[1])


# speedup vs baseline: 10.6294x; 1.0978x over previous
import jax
import jax.numpy as jnp
from jax import lax
from jax.experimental import pallas as pl
from jax.experimental.pallas import tpu as pltpu
from jax.experimental.pallas import tpu_sc as plsc

N_NODES = 10000
E_EDGES = 320000
D = 128

NUM_CORES = 2
NUM_SUBCORES = 16
CHUNK = 125
EDGES_PER_TILE = E_EDGES // NUM_SUBCORES
NCHUNK = EDGES_PER_TILE // CHUNK
NPAIR = NCHUNK // 2
ROWS_PER_TILE = N_NODES // NUM_SUBCORES
CNT_W = 8


def _sc_aggregate_body(x_hbm, sidx_hbm, didx_hbm, zrow_hbm, zcnt_hbm,
                       ones_hbm, acc_hbm, cnt_hbm,
                       srca, srcb, dsta, dstb, msga, msgb, ones_v,
                       sem_ga, sem_gb, sem_sa, sem_sb, sem_ca, sem_cb,
                       sem_isa, sem_isb, sem_ida, sem_idb,
                       acc_sh, cnt_sh):
    c = lax.axis_index("c")
    s = lax.axis_index("s")
    rows = pl.ds(s * ROWS_PER_TILE, ROWS_PER_TILE)

    pltpu.sync_copy(zrow_hbm, acc_sh.at[rows, :])
    pltpu.sync_copy(zcnt_hbm, cnt_sh.at[rows, :])
    pltpu.sync_copy(ones_hbm, ones_v)
    plsc.subcore_barrier()

    pltpu.sync_copy(sidx_hbm.at[c, s, 0], srca)
    pltpu.async_copy(x_hbm.at[srca], msga, sem_ga)
    pltpu.async_copy(didx_hbm.at[c, s, 0], dsta, sem_ida)
    pltpu.async_copy(sidx_hbm.at[c, s, 1], srcb, sem_isb)

    def step(i, carry):
        j0 = 2 * i
        pltpu.make_async_copy(x_hbm.at[srca], msga, sem_ga).wait()
        pltpu.make_async_copy(didx_hbm.at[c, s, j0], dsta, sem_ida).wait()
        pltpu.async_copy(msga, acc_sh.at[dsta], sem_sa, add=True)
        pltpu.async_copy(ones_v, cnt_sh.at[dsta], sem_ca, add=True)

        @pl.when(i + 1 < NPAIR)
        def _():
            pltpu.async_copy(sidx_hbm.at[c, s, j0 + 2], srca, sem_isa)

        @pl.when(i > 0)
        def _():
            pltpu.make_async_copy(msgb, acc_sh.at[dstb], sem_sb).wait()
            pltpu.make_async_copy(ones_v, cnt_sh.at[dstb], sem_cb).wait()

        pltpu.make_async_copy(sidx_hbm.at[c, s, j0 + 1], srcb, sem_isb).wait()
        pltpu.async_copy(x_hbm.at[srcb], msgb, sem_gb)
        pltpu.async_copy(didx_hbm.at[c, s, j0 + 1], dstb, sem_idb)

        pltpu.make_async_copy(x_hbm.at[srcb], msgb, sem_gb).wait()
        pltpu.make_async_copy(didx_hbm.at[c, s, j0 + 1], dstb, sem_idb).wait()
        pltpu.async_copy(msgb, acc_sh.at[dstb], sem_sb, add=True)
        pltpu.async_copy(ones_v, cnt_sh.at[dstb], sem_cb, add=True)

        @pl.when(i + 1 < NPAIR)
        def _():
            pltpu.async_copy(sidx_hbm.at[c, s, j0 + 3], srcb, sem_isb)
            pltpu.make_async_copy(msga, acc_sh.at[dsta], sem_sa).wait()
            pltpu.make_async_copy(ones_v, cnt_sh.at[dsta], sem_ca).wait()
            pltpu.make_async_copy(sidx_hbm.at[c, s, j0 + 2], srca, sem_isa).wait()
            pltpu.async_copy(x_hbm.at[srca], msga, sem_ga)
            pltpu.async_copy(didx_hbm.at[c, s, j0 + 2], dsta, sem_ida)

        return carry

    lax.fori_loop(0, NPAIR, step, 0)
    pltpu.make_async_copy(msga, acc_sh.at[dsta], sem_sa).wait()
    pltpu.make_async_copy(ones_v, cnt_sh.at[dsta], sem_ca).wait()
    pltpu.make_async_copy(msgb, acc_sh.at[dstb], sem_sb).wait()
    pltpu.make_async_copy(ones_v, cnt_sh.at[dstb], sem_cb).wait()
    plsc.subcore_barrier()

    pltpu.sync_copy(acc_sh.at[rows, :], acc_hbm.at[c, rows, :])
    pltpu.sync_copy(cnt_sh.at[rows, :], cnt_hbm.at[c, rows, :])


def _tc_linear_body(acc_ref, cnt_ref, wt_ref, b_ref, out_ref):
    cnt = cnt_ref[0, :, 0:1]
    recip = 1.0 / jnp.maximum(cnt, 1.0)
    gate = jnp.minimum(cnt, 1.0)
    x = acc_ref[0] * recip
    out_ref[0] = (jnp.dot(x, wt_ref[0], preferred_element_type=jnp.float32)
                  + gate * b_ref[0])


def kernel(x_user, x_item, edge_index_follows, edge_index_clicks,
           W_follows, b_follows, W_clicks, b_clicks):
    del x_item

    sidx = jnp.stack([
        edge_index_follows[0].reshape(NUM_SUBCORES, NCHUNK, CHUNK),
        edge_index_clicks[0].reshape(NUM_SUBCORES, NCHUNK, CHUNK),
    ])
    didx = jnp.stack([
        edge_index_follows[1].reshape(NUM_SUBCORES, NCHUNK, CHUNK),
        edge_index_clicks[1].reshape(NUM_SUBCORES, NCHUNK, CHUNK),
    ])
    zrow = jnp.zeros((ROWS_PER_TILE, D), jnp.float32)
    zcnt = jnp.zeros((ROWS_PER_TILE, CNT_W), jnp.float32)
    ones = jnp.ones((CHUNK, CNT_W), jnp.float32)

    mesh = plsc.VectorSubcoreMesh(core_axis_name="c", subcore_axis_name="s",
                                  num_cores=NUM_CORES,
                                  num_subcores=NUM_SUBCORES)
    acc, cnt = pl.kernel(
        _sc_aggregate_body,
        out_type=[
            jax.ShapeDtypeStruct((NUM_CORES, N_NODES, D), jnp.float32),
            jax.ShapeDtypeStruct((NUM_CORES, N_NODES, CNT_W), jnp.float32),
        ],
        mesh=mesh,
        compiler_params=pltpu.CompilerParams(use_tc_tiling_on_sc=False),
        scratch_types=[
            pltpu.VMEM((CHUNK,), jnp.int32),
            pltpu.VMEM((CHUNK,), jnp.int32),
            pltpu.VMEM((CHUNK,), jnp.int32),
            pltpu.VMEM((CHUNK,), jnp.int32),
            pltpu.VMEM((CHUNK, D), jnp.float32),
            pltpu.VMEM((CHUNK, D), jnp.float32),
            pltpu.VMEM((CHUNK, CNT_W), jnp.float32),
        ] + [pltpu.SemaphoreType.DMA] * 10 + [
            pltpu.VMEM_SHARED((N_NODES, D), jnp.float32),
            pltpu.VMEM_SHARED((N_NODES, CNT_W), jnp.float32),
        ],
    )(x_user, sidx, didx, zrow, zcnt, ones)

    wt = jnp.stack([W_follows.T, W_clicks.T])
    b = jnp.stack([b_follows, b_clicks]).reshape(NUM_CORES, 1, D)

    ROW_BLK = 1000
    out = pl.pallas_call(
        _tc_linear_body,
        grid=(NUM_CORES, N_NODES // ROW_BLK),
        in_specs=[
            pl.BlockSpec((1, ROW_BLK, D), lambda t, m: (t, m, 0)),
            pl.BlockSpec((1, ROW_BLK, CNT_W), lambda t, m: (t, m, 0)),
            pl.BlockSpec((1, D, D), lambda t, m: (t, 0, 0)),
            pl.BlockSpec((1, 1, D), lambda t, m: (t, 0, 0)),
        ],
        out_specs=pl.BlockSpec((1, ROW_BLK, D), lambda t, m: (t, m, 0)),
        out_shape=jax.ShapeDtypeStruct((NUM_CORES, N_NODES, D), jnp.float32),
    )(acc, cnt, wt, b)

    return (out[0], out[1])


# speedup vs baseline: 10.9513x; 1.0303x over previous
import jax
import jax.numpy as jnp
from jax import lax
from jax.experimental import pallas as pl
from jax.experimental.pallas import tpu as pltpu
from jax.experimental.pallas import tpu_sc as plsc

N_NODES = 10000
E_EDGES = 320000
D = 128

NUM_CORES = 2
NUM_SUBCORES = 16
CHUNK = 125
EDGES_PER_TILE = E_EDGES // NUM_SUBCORES
NCHUNK = EDGES_PER_TILE // CHUNK
NPAIR = NCHUNK // 2
ROWS_PER_TILE = N_NODES // NUM_SUBCORES
CNT_W = 8


def _sc_aggregate_body(x_hbm, sidx_hbm, didx_hbm, zrow_hbm, zcnt_hbm,
                       ones_hbm, acc_hbm, cnt_hbm,
                       srca, srcb, dsta, dstb, msga, msgb, ones_v,
                       sem_ga, sem_gb, sem_sa, sem_sb, sem_ca, sem_cb,
                       sem_isa, sem_isb, sem_ida, sem_idb,
                       acc_sh, cnt_sh):
    c = lax.axis_index("c")
    s = lax.axis_index("s")
    rows = pl.ds(s * ROWS_PER_TILE, ROWS_PER_TILE)

    pltpu.sync_copy(zrow_hbm, acc_sh.at[rows, :])
    pltpu.sync_copy(zcnt_hbm, cnt_sh.at[rows, :])
    pltpu.sync_copy(ones_hbm, ones_v)
    plsc.subcore_barrier()

    pltpu.sync_copy(sidx_hbm.at[c, s, 0], srca)
    pltpu.async_copy(x_hbm.at[srca], msga, sem_ga)
    pltpu.async_copy(didx_hbm.at[c, s, 0], dsta, sem_ida)
    pltpu.async_copy(sidx_hbm.at[c, s, 1], srcb, sem_isb)

    def step(i, carry):
        j0 = 2 * i
        pltpu.make_async_copy(x_hbm.at[srca], msga, sem_ga).wait()
        pltpu.make_async_copy(didx_hbm.at[c, s, j0], dsta, sem_ida).wait()
        pltpu.async_copy(msga, acc_sh.at[dsta], sem_sa, add=True)
        pltpu.async_copy(ones_v, cnt_sh.at[dsta], sem_ca, add=True)

        @pl.when(i + 1 < NPAIR)
        def _():
            pltpu.async_copy(sidx_hbm.at[c, s, j0 + 2], srca, sem_isa)

        @pl.when(i > 0)
        def _():
            pltpu.make_async_copy(msgb, acc_sh.at[dstb], sem_sb).wait()
            pltpu.make_async_copy(ones_v, cnt_sh.at[dstb], sem_cb).wait()

        pltpu.make_async_copy(sidx_hbm.at[c, s, j0 + 1], srcb, sem_isb).wait()
        pltpu.async_copy(x_hbm.at[srcb], msgb, sem_gb)
        pltpu.async_copy(didx_hbm.at[c, s, j0 + 1], dstb, sem_idb)

        pltpu.make_async_copy(x_hbm.at[srcb], msgb, sem_gb).wait()
        pltpu.make_async_copy(didx_hbm.at[c, s, j0 + 1], dstb, sem_idb).wait()
        pltpu.async_copy(msgb, acc_sh.at[dstb], sem_sb, add=True)
        pltpu.async_copy(ones_v, cnt_sh.at[dstb], sem_cb, add=True)

        @pl.when(i + 1 < NPAIR)
        def _():
            pltpu.async_copy(sidx_hbm.at[c, s, j0 + 3], srcb, sem_isb)
            pltpu.make_async_copy(msga, acc_sh.at[dsta], sem_sa).wait()
            pltpu.make_async_copy(ones_v, cnt_sh.at[dsta], sem_ca).wait()
            pltpu.make_async_copy(sidx_hbm.at[c, s, j0 + 2], srca, sem_isa).wait()
            pltpu.async_copy(x_hbm.at[srca], msga, sem_ga)
            pltpu.async_copy(didx_hbm.at[c, s, j0 + 2], dsta, sem_ida)

        return carry

    lax.fori_loop(0, NPAIR, step, 0)
    pltpu.make_async_copy(msga, acc_sh.at[dsta], sem_sa).wait()
    pltpu.make_async_copy(ones_v, cnt_sh.at[dsta], sem_ca).wait()
    pltpu.make_async_copy(msgb, acc_sh.at[dstb], sem_sb).wait()
    pltpu.make_async_copy(ones_v, cnt_sh.at[dstb], sem_cb).wait()
    plsc.subcore_barrier()

    pltpu.sync_copy(acc_sh.at[rows, :], acc_hbm.at[c, rows, :])
    pltpu.sync_copy(cnt_sh.at[rows, :], cnt_hbm.at[c, rows, :])


def _tc_linear_body(accf_ref, cntf_ref, wtf_ref, bf_ref,
                    accc_ref, cntc_ref, wtc_ref, bc_ref,
                    outu_ref, outi_ref):
    cntf = cntf_ref[0, :, 0:1]
    outu_ref[...] = (
        jnp.dot(accf_ref[0] * (1.0 / jnp.maximum(cntf, 1.0)), wtf_ref[...],
                preferred_element_type=jnp.float32)
        + jnp.minimum(cntf, 1.0) * bf_ref[...])
    cntc = cntc_ref[0, :, 0:1]
    outi_ref[...] = (
        jnp.dot(accc_ref[0] * (1.0 / jnp.maximum(cntc, 1.0)), wtc_ref[...],
                preferred_element_type=jnp.float32)
        + jnp.minimum(cntc, 1.0) * bc_ref[...])


def kernel(x_user, x_item, edge_index_follows, edge_index_clicks,
           W_follows, b_follows, W_clicks, b_clicks):
    del x_item

    sidx = jnp.stack([
        edge_index_follows[0].reshape(NUM_SUBCORES, NCHUNK, CHUNK),
        edge_index_clicks[0].reshape(NUM_SUBCORES, NCHUNK, CHUNK),
    ])
    didx = jnp.stack([
        edge_index_follows[1].reshape(NUM_SUBCORES, NCHUNK, CHUNK),
        edge_index_clicks[1].reshape(NUM_SUBCORES, NCHUNK, CHUNK),
    ])
    zrow = jnp.zeros((ROWS_PER_TILE, D), jnp.float32)
    zcnt = jnp.zeros((ROWS_PER_TILE, CNT_W), jnp.float32)
    ones = jnp.ones((CHUNK, CNT_W), jnp.float32)

    mesh = plsc.VectorSubcoreMesh(core_axis_name="c", subcore_axis_name="s",
                                  num_cores=NUM_CORES,
                                  num_subcores=NUM_SUBCORES)
    acc, cnt = pl.kernel(
        _sc_aggregate_body,
        out_type=[
            jax.ShapeDtypeStruct((NUM_CORES, N_NODES, D), jnp.float32),
            jax.ShapeDtypeStruct((NUM_CORES, N_NODES, CNT_W), jnp.float32),
        ],
        mesh=mesh,
        compiler_params=pltpu.CompilerParams(use_tc_tiling_on_sc=False),
        scratch_types=[
            pltpu.VMEM((CHUNK,), jnp.int32),
            pltpu.VMEM((CHUNK,), jnp.int32),
            pltpu.VMEM((CHUNK,), jnp.int32),
            pltpu.VMEM((CHUNK,), jnp.int32),
            pltpu.VMEM((CHUNK, D), jnp.float32),
            pltpu.VMEM((CHUNK, D), jnp.float32),
            pltpu.VMEM((CHUNK, CNT_W), jnp.float32),
        ] + [pltpu.SemaphoreType.DMA] * 10 + [
            pltpu.VMEM_SHARED((N_NODES, D), jnp.float32),
            pltpu.VMEM_SHARED((N_NODES, CNT_W), jnp.float32),
        ],
    )(x_user, sidx, didx, zrow, zcnt, ones)

    ROW_BLK = 1000
    grid = (N_NODES // ROW_BLK,)
    accf_spec = pl.BlockSpec((1, ROW_BLK, D), lambda m: (0, m, 0))
    accc_spec = pl.BlockSpec((1, ROW_BLK, D), lambda m: (1, m, 0))
    cntf_spec = pl.BlockSpec((1, ROW_BLK, CNT_W), lambda m: (0, m, 0))
    cntc_spec = pl.BlockSpec((1, ROW_BLK, CNT_W), lambda m: (1, m, 0))
    w_spec = pl.BlockSpec((D, D), lambda m: (0, 0))
    b_spec = pl.BlockSpec((1, D), lambda m: (0, 0))
    out_spec = pl.BlockSpec((ROW_BLK, D), lambda m: (m, 0))
    out_user, out_item = pl.pallas_call(
        _tc_linear_body,
        grid=grid,
        in_specs=[accf_spec, cntf_spec, w_spec, b_spec,
                  accc_spec, cntc_spec, w_spec, b_spec],
        out_specs=[out_spec, out_spec],
        out_shape=[jax.ShapeDtypeStruct((N_NODES, D), jnp.float32),
                   jax.ShapeDtypeStruct((N_NODES, D), jnp.float32)],
    )(acc, cnt, W_follows.T, b_follows.reshape(1, D),
      acc, cnt, W_clicks.T, b_clicks.reshape(1, D))

    return (out_user, out_item)
